# baseline (device time: 30415 ns/iter reference)
import jax
import jax.numpy as jnp
from jax import lax
from jax.experimental import pallas as pl
from jax.experimental.pallas import tpu as pltpu

N_DEV = 8
N_GLOBAL = 8192
EPS = 1e-5
PC = 128
MB = 512


def _row_mask(mb):
    r0 = lax.broadcasted_iota(jnp.int32, (mb, PC), 0)
    c0 = lax.broadcasted_iota(jnp.int32, (mb, PC), 1)
    return jnp.bitwise_and(r0, PC - 1) == c0


def _allreduce_sumsq(x):
    m, n_per = x.shape
    pr, nb, pb = m // PC, m // MB, MB // PC

    def body(x_ref, out_ref, comm_ref, send_sems, recv_sems):
        i = pl.program_id(0)
        my = lax.axis_index("i")

        xx = x_ref[:, :]
        rowsum = jnp.sum(xx * xx, axis=1, keepdims=True)
        mask = _row_mask(MB)
        d = jnp.where(mask, jnp.broadcast_to(rowsum, (MB, PC)), 0.0)
        bi = lax.broadcasted_iota(jnp.int32, (pb, MB), 0)
        br = lax.broadcasted_iota(jnp.int32, (pb, MB), 1)
        blk = (br // PC == bi).astype(jnp.float32)
        comm_ref[0, pl.ds(i * pb, pb), :] = jnp.dot(
            blk, d, preferred_element_type=jnp.float32
        )

        @pl.when(i == nb - 1)
        def _comm():
            bar = pltpu.get_barrier_semaphore()
            for dd in range(1, N_DEV):
                peer = (my + dd) % N_DEV
                pl.semaphore_signal(
                    bar, inc=1, device_id=(peer,),
                    device_id_type=pl.DeviceIdType.MESH,
                )
            pl.semaphore_wait(bar, N_DEV - 1)

            rdmas = []
            for dd in range(1, N_DEV):
                peer = (my + dd) % N_DEV
                rdma = pltpu.make_async_remote_copy(
                    src_ref=comm_ref.at[0],
                    dst_ref=comm_ref.at[dd],
                    send_sem=send_sems.at[dd],
                    recv_sem=recv_sems.at[dd],
                    device_id=(peer,),
                    device_id_type=pl.DeviceIdType.MESH,
                )
                rdma.start()
                rdmas.append(rdma)

            total = comm_ref[0, :, :]
            for dd in range(1, N_DEV):
                rdmas[dd - 1].wait_recv()
                total = total + comm_ref[dd, :, :]
            for dd in range(1, N_DEV):
                rdmas[dd - 1].wait_send()
            out_ref[:, :] = total

    return pl.pallas_call(
        body,
        grid=(nb,),
        out_shape=jax.ShapeDtypeStruct((pr, PC), jnp.float32),
        in_specs=[pl.BlockSpec((MB, n_per), lambda i: (i, 0))],
        out_specs=pl.BlockSpec((pr, PC), lambda i: (0, 0)),
        scratch_shapes=[
            pltpu.VMEM((N_DEV, pr, PC), jnp.float32),
            pltpu.SemaphoreType.DMA((N_DEV,)),
            pltpu.SemaphoreType.DMA((N_DEV,)),
        ],
        compiler_params=pltpu.CompilerParams(
            collective_id=0, vmem_limit_bytes=64 * 1024 * 1024
        ),
    )(x)


def _scale(x, total, gamma2):
    m, n_per = x.shape
    nb, pb = m // MB, MB // PC

    def body(x_ref, t_ref, g_ref, out_ref):
        i = pl.program_id(0)
        t_blk = t_ref[pl.ds(i * pb, pb), :]
        br2 = lax.broadcasted_iota(jnp.int32, (MB, pb), 0)
        bi2 = lax.broadcasted_iota(jnp.int32, (MB, pb), 1)
        blk_t = (br2 // PC == bi2).astype(jnp.float32)
        t2 = jnp.dot(blk_t, t_blk, preferred_element_type=jnp.float32)
        mask = _row_mask(MB)
        tot_col = jnp.sum(jnp.where(mask, t2, 0.0), axis=1, keepdims=True)
        rstd = lax.rsqrt(tot_col / N_GLOBAL + EPS)
        out_ref[:, :] = x_ref[:, :] * rstd * g_ref[:, :]

    return pl.pallas_call(
        body,
        grid=(nb,),
        out_shape=jax.ShapeDtypeStruct((m, n_per), x.dtype),
        in_specs=[
            pl.BlockSpec((MB, n_per), lambda i: (i, 0)),
            pl.BlockSpec((m // PC, PC), lambda i: (0, 0)),
            pl.BlockSpec((1, n_per), lambda i: (0, 0)),
        ],
        out_specs=pl.BlockSpec((MB, n_per), lambda i: (i, 0)),
        compiler_params=pltpu.CompilerParams(
            vmem_limit_bytes=64 * 1024 * 1024
        ),
    )(x, total, gamma2)


def kernel(x, gamma):
    m, n_per = x.shape
    assert m % MB == 0 and MB % PC == 0
    total = _allreduce_sumsq(x)
    return _scale(x, total, gamma.reshape(1, n_per))


# device time: 27637 ns/iter; 1.1005x vs baseline; 1.1005x over previous
import jax
import jax.numpy as jnp
from jax import lax
from jax.experimental import pallas as pl
from jax.experimental.pallas import tpu as pltpu

N_DEV = 8
N_GLOBAL = 8192
EPS = 1e-5
PC = 128


def _pack_mats(m):
    pr = m // PC
    r0 = lax.broadcasted_iota(jnp.int32, (m, PC), 0)
    c0 = lax.broadcasted_iota(jnp.int32, (m, PC), 1)
    mask = jnp.bitwise_and(r0, PC - 1) == c0
    bi = lax.broadcasted_iota(jnp.int32, (pr, m), 0)
    br = lax.broadcasted_iota(jnp.int32, (pr, m), 1)
    blk = (br // PC == bi).astype(jnp.float32)
    br2 = lax.broadcasted_iota(jnp.int32, (m, pr), 0)
    bi2 = lax.broadcasted_iota(jnp.int32, (m, pr), 1)
    blk_t = (br2 // PC == bi2).astype(jnp.float32)
    return mask, blk, blk_t


def _allreduce_sumsq(x):
    m, n_per = x.shape
    pr = m // PC

    def body(x_ref, out_ref, comm_ref, send_sems, recv_sems):
        my = lax.axis_index("i")

        bar = pltpu.get_barrier_semaphore()
        for dd in range(1, N_DEV):
            peer = (my + dd) % N_DEV
            pl.semaphore_signal(
                bar, inc=1, device_id=(peer,),
                device_id_type=pl.DeviceIdType.MESH,
            )

        mask, blk, _ = _pack_mats(m)

        xx = x_ref[:, :]
        rowsum = jnp.sum(xx * xx, axis=1, keepdims=True)
        d = jnp.where(mask, jnp.broadcast_to(rowsum, (m, PC)), 0.0)
        comm_ref[0, :, :] = jnp.dot(blk, d, preferred_element_type=jnp.float32)

        pl.semaphore_wait(bar, N_DEV - 1)

        rdmas = []
        for dd in range(1, N_DEV):
            peer = (my + dd) % N_DEV
            rdma = pltpu.make_async_remote_copy(
                src_ref=comm_ref.at[0],
                dst_ref=comm_ref.at[dd],
                send_sem=send_sems.at[dd],
                recv_sem=recv_sems.at[dd],
                device_id=(peer,),
                device_id_type=pl.DeviceIdType.MESH,
            )
            rdma.start()
            rdmas.append(rdma)

        total = comm_ref[0, :, :]
        for dd in range(1, N_DEV):
            rdmas[dd - 1].wait_recv()
            total = total + comm_ref[dd, :, :]
        for dd in range(1, N_DEV):
            rdmas[dd - 1].wait_send()
        out_ref[:, :] = total

    return pl.pallas_call(
        body,
        out_shape=jax.ShapeDtypeStruct((pr, PC), jnp.float32),
        in_specs=[pl.BlockSpec(memory_space=pltpu.VMEM)],
        out_specs=pl.BlockSpec(memory_space=pltpu.VMEM),
        scratch_shapes=[
            pltpu.VMEM((N_DEV, pr, PC), jnp.float32),
            pltpu.SemaphoreType.DMA((N_DEV,)),
            pltpu.SemaphoreType.DMA((N_DEV,)),
        ],
        compiler_params=pltpu.CompilerParams(
            collective_id=0, vmem_limit_bytes=64 * 1024 * 1024
        ),
    )(x)


def _scale(x, total, gamma2):
    m, n_per = x.shape

    def body(x_ref, t_ref, g_ref, out_ref):
        mask, _, blk_t = _pack_mats(m)
        t2 = jnp.dot(blk_t, t_ref[:, :], preferred_element_type=jnp.float32)
        tot_col = jnp.sum(jnp.where(mask, t2, 0.0), axis=1, keepdims=True)
        rstd = lax.rsqrt(tot_col / N_GLOBAL + EPS)
        out_ref[:, :] = x_ref[:, :] * rstd * g_ref[:, :]

    return pl.pallas_call(
        body,
        out_shape=jax.ShapeDtypeStruct((m, n_per), x.dtype),
        in_specs=[
            pl.BlockSpec(memory_space=pltpu.VMEM),
            pl.BlockSpec(memory_space=pltpu.VMEM),
            pl.BlockSpec(memory_space=pltpu.VMEM),
        ],
        out_specs=pl.BlockSpec(memory_space=pltpu.VMEM),
        compiler_params=pltpu.CompilerParams(
            vmem_limit_bytes=64 * 1024 * 1024
        ),
    )(x, total, gamma2)


def kernel(x, gamma):
    m, n_per = x.shape
    assert m % PC == 0
    total = _allreduce_sumsq(x)
    return _scale(x, total, gamma.reshape(1, n_per))
